# Initial kernel scaffold; baseline (speedup 1.0000x reference)
import jax
import jax.numpy as jnp
from jax.experimental import pallas as pl


def kernel(x, enc_w1, enc_b1, enc_w2, enc_b2, enc_w3, enc_b3, codebook, dec_w1, dec_b1, dec_w2, dec_b2, dec_w3, dec_b3):
    raise NotImplementedError("write your pallas kernel here")


# speedup vs baseline: 1.8394x; 1.8394x over previous
import functools

import jax
import jax.numpy as jnp
from jax import lax
from jax.experimental import pallas as pl
from jax.experimental.pallas import tpu as pltpu
from jax.experimental.pallas import tpu_sc as plsc

F32 = jnp.float32
BF16 = jnp.bfloat16
NUM_EMB = 8192
LATENT = 64
CC = 0.25
B = 4
M = 2048
TS = 256

_HIGH = lax.Precision.HIGHEST


def _dot(a, b, precision=None):
    return lax.dot_general(a, b, (((1,), (0,)), ((), ())),
                           preferred_element_type=F32, precision=precision)


def _enc_body(x_ref, w1_ref, b1_ref, w2_ref, b2_ref, w3_ref, b3_ref,
              z_ref, zb_ref, rn_ref):
    X = x_ref[0]
    b1 = b1_ref[0]
    b2 = b2_ref[0]
    b3 = b3_ref[0]

    taps1 = [
        [(7, -1), (0, 0), (1, 0), (2, 0)],
        [(1, 0), (2, 0), (3, 0), (4, 0)],
        [(3, 0), (4, 0), (5, 0), (6, 0)],
        [(5, 0), (6, 0), (7, 0), (0, 1)],
    ]
    rows1 = jax.lax.broadcasted_iota(jnp.int32, (2050, 1), 0)
    edge1 = (rows1 == 0) | (rows1 == 2049)
    h1b = []
    for p in range(4):
        cols = [X[1 + d:2051 + d, r:r + 1] for (r, d) in taps1[p]]
        patch = jnp.concatenate(cols, axis=1).astype(F32)
        pre = _dot(patch, w1_ref[...]) + b1[None, :]
        h1 = jnp.maximum(pre, 0.0)
        h1 = jnp.where(edge1, 0.0, h1)
        h1b.append(h1.astype(BF16))

    taps2 = [
        [(3, -1), (0, 0), (1, 0), (2, 0)],
        [(1, 0), (2, 0), (3, 0), (0, 1)],
    ]
    zrow256 = jnp.zeros((1, 256), BF16)
    h2b = []
    for q in range(2):
        blocks = [h1b[p][1 + d:2049 + d, :] for (p, d) in taps2[q]]
        patch = jnp.concatenate(blocks, axis=1)
        pre = _dot(patch, w2_ref[...]) + b2[None, :]
        h2 = jnp.maximum(pre, 0.0)
        h2b.append(jnp.concatenate([zrow256, h2.astype(BF16), zrow256], axis=0))

    taps3 = [(1, -1), (0, 0), (1, 0), (0, 1)]
    blocks = [h2b[p][1 + d:2049 + d, :] for (p, d) in taps3]
    w3full = w3_ref[...]
    ds = [_dot(blocks[t], w3full[256 * t:256 * (t + 1), :]) for t in range(4)]
    pre = ((ds[0] + ds[1]) + ds[2]) + ds[3] + b3[None, :]
    zT = jnp.maximum(pre, 0.0)

    z_ref[0] = zT
    zb_ref[0] = zT.astype(BF16)
    rn_ref[0, 0, :] = jnp.sum(zT * zT, axis=1)


def _encoder(xph, w1k, b1, w2k, b2, w3k, b3):
    return pl.pallas_call(
        _enc_body,
        grid=(B,),
        in_specs=[
            pl.BlockSpec((1, 2052, 8), lambda b: (b, 0, 0)),
            pl.BlockSpec((4, 128), lambda b: (0, 0)),
            pl.BlockSpec((1, 128), lambda b: (0, 0)),
            pl.BlockSpec((512, 256), lambda b: (0, 0)),
            pl.BlockSpec((1, 256), lambda b: (0, 0)),
            pl.BlockSpec((1024, 64), lambda b: (0, 0)),
            pl.BlockSpec((1, 64), lambda b: (0, 0)),
        ],
        out_specs=[
            pl.BlockSpec((1, M, 64), lambda b: (b, 0, 0)),
            pl.BlockSpec((1, M, 64), lambda b: (b, 0, 0)),
            pl.BlockSpec((1, 1, M), lambda b: (b, 0, 0)),
        ],
        out_shape=[
            jax.ShapeDtypeStruct((B, M, 64), F32),
            jax.ShapeDtypeStruct((B, M, 64), BF16),
            jax.ShapeDtypeStruct((B, 1, M), F32),
        ],
    )(xph, w1k, b1, w2k, b2, w3k, b3)


def _dist_body(flatb_ref, cbt_ref, rn_ref, idx_ref, cn_scr):
    i = pl.program_id(0)

    @pl.when(i == 0)
    def _():
        cbt = cbt_ref[...]
        cn_scr[...] = jnp.sum(cbt * cbt, axis=0, keepdims=True)

    flat = flatb_ref[...].astype(F32)
    dot = _dot(flat, cbt_ref[...])
    d = (rn_ref[...] + cn_scr[...]) - 2.0 * dot
    mn = jnp.min(d, axis=1)
    codes = jax.lax.broadcasted_iota(jnp.int32, d.shape, 1)
    idx = jnp.min(jnp.where(d == mn[:, None], codes, NUM_EMB), axis=1)
    idx_ref[0, 0, :] = idx


def _distance_argmin(flatb, cbt, rn2):
    nt = (B * M) // TS
    return pl.pallas_call(
        _dist_body,
        grid=(nt,),
        in_specs=[
            pl.BlockSpec((TS, 64), lambda i: (i, 0)),
            pl.BlockSpec((64, NUM_EMB), lambda i: (0, 0)),
            pl.BlockSpec((TS, 1), lambda i: (i, 0)),
        ],
        out_specs=pl.BlockSpec((1, 1, TS), lambda i: (i, 0, 0)),
        out_shape=jax.ShapeDtypeStruct((nt, 1, TS), jnp.int32),
        scratch_shapes=[pltpu.VMEM((1, NUM_EMB), F32)],
    )(flatb, cbt, rn2)


_BPW = (B * M) // 32


def _sc_gather(table, idx):
    mesh = plsc.VectorSubcoreMesh(core_axis_name="c", subcore_axis_name="s")

    @functools.partial(
        pl.kernel,
        out_type=jax.ShapeDtypeStruct((B * M, 128), F32),
        mesh=mesh,
        scratch_types=[
            pltpu.VMEM((_BPW,), jnp.int32),
            pltpu.VMEM((_BPW, 128), F32),
            pltpu.SemaphoreType.DMA,
        ],
    )
    def body(table_hbm, idx_hbm, out_hbm, idx_v, rows_v, sem):
        wid = lax.axis_index("s") * 2 + lax.axis_index("c")
        base = wid * _BPW
        pltpu.sync_copy(idx_hbm.at[pl.ds(base, _BPW)], idx_v)
        pltpu.async_copy(table_hbm.at[idx_v], rows_v, sem).wait()
        pltpu.sync_copy(rows_v, out_hbm.at[pl.ds(base, _BPW)])

    return body(table, idx)


def _dec_body(q_ref, z_ref, we1_ref, wo1_ref, db1_ref, w2s_ref, db2_ref,
              w3s_ref, db3_ref, qout_ref, rec_ref, loss_ref):
    b = pl.program_id(0)
    zT = z_ref[0]
    sub = q_ref[0] - zT
    qout = zT + sub
    qout_ref[0] = qout

    part = jnp.sum(sub * sub, keepdims=True).reshape(1, 1)

    @pl.when(b == 0)
    def _():
        loss_ref[...] = jnp.zeros((1, 1), F32)
    loss_ref[...] += part

    @pl.when(b == B - 1)
    def _():
        mean = loss_ref[...] / (B * M * LATENT)
        loss_ref[...] = mean + CC * mean

    db1 = db1_ref[0]
    db2 = db2_ref[0]

    qb = qout.astype(BF16)
    zrow64 = jnp.zeros((1, 64), BF16)
    qp = jnp.concatenate([zrow64, qb, zrow64], axis=0)

    patch_e = jnp.concatenate([qp[1:2049], qp[0:2048]], axis=1)
    d1e = jnp.maximum(_dot(patch_e, we1_ref[...]) + db1[None, :], 0.0)
    patch_o = jnp.concatenate([qp[1:2049], qp[2:2050]], axis=1)
    d1o = jnp.maximum(_dot(patch_o, wo1_ref[...]) + db1[None, :], 0.0)

    zrow256 = jnp.zeros((1, 256), BF16)
    d1ep = jnp.concatenate([zrow256, d1e.astype(BF16), zrow256], axis=0)
    d1op = jnp.concatenate([zrow256, d1o.astype(BF16), zrow256], axis=0)

    taps = [
        ((d1ep, 0), (d1op, -1)),
        ((d1op, 0), (d1ep, 0)),
        ((d1op, 0), (d1ep, 0)),
        ((d1ep, 1), (d1op, 0)),
    ]
    d2 = []
    for s in range(4):
        (a, da), (c, dc) = taps[s]
        patch = jnp.concatenate(
            [a[1 + da:2049 + da], c[1 + dc:2049 + dc]], axis=1)
        pre = _dot(patch, w2s_ref[s]) + db2[None, :]
        d2.append(jnp.maximum(pre, 0.0).astype(BF16))

    zrow128 = jnp.zeros((1, 128), BF16)
    d2p = [jnp.concatenate([zrow128, t, zrow128], axis=0) for t in d2]

    patch = jnp.concatenate(
        [d2p[0][1:2049], d2p[1][1:2049], d2p[2][1:2049], d2p[3][1:2049],
         d2p[3][0:2048], d2p[0][2:2050]], axis=1)
    rec = jnp.tanh(_dot(patch, w3s_ref[...]) + db3_ref[...])
    rec_ref[0] = rec


def _decoder(q, zT, we1, wo1, db1, w2s, db2, w3s, db3):
    return pl.pallas_call(
        _dec_body,
        grid=(B,),
        in_specs=[
            pl.BlockSpec((1, M, 64), lambda b: (b, 0, 0)),
            pl.BlockSpec((1, M, 64), lambda b: (b, 0, 0)),
            pl.BlockSpec((128, 256), lambda b: (0, 0)),
            pl.BlockSpec((128, 256), lambda b: (0, 0)),
            pl.BlockSpec((1, 256), lambda b: (0, 0)),
            pl.BlockSpec((4, 512, 128), lambda b: (0, 0, 0)),
            pl.BlockSpec((1, 128), lambda b: (0, 0)),
            pl.BlockSpec((768, 8), lambda b: (0, 0)),
            pl.BlockSpec((1, 1), lambda b: (0, 0)),
        ],
        out_specs=[
            pl.BlockSpec((1, M, 64), lambda b: (b, 0, 0)),
            pl.BlockSpec((1, M, 8), lambda b: (b, 0, 0)),
            pl.BlockSpec((1, 1), lambda b: (0, 0)),
        ],
        out_shape=[
            jax.ShapeDtypeStruct((B, M, 64), F32),
            jax.ShapeDtypeStruct((B, M, 8), F32),
            jax.ShapeDtypeStruct((1, 1), F32),
        ],
    )(q, zT, we1, wo1, db1, w2s, db2, w3s, db3)


def kernel(x, enc_w1, enc_b1, enc_w2, enc_b2, enc_w3, enc_b3, codebook,
           dec_w1, dec_b1, dec_w2, dec_b2, dec_w3, dec_b3):
    xb = x.reshape(B, 16384).astype(BF16)
    xpad = jnp.pad(xb, ((0, 0), (16, 16)))
    xph = xpad.reshape(B, 2052, 8)

    w1k = enc_w1.reshape(128, 4).T
    w2k = enc_w2.transpose(2, 1, 0).reshape(512, 256).astype(BF16)
    w3k = enc_w3.transpose(2, 1, 0).reshape(1024, 64).astype(BF16)

    zT, zTb, rn = _encoder(xph, w1k, enc_b1.reshape(1, 128),
                           w2k, enc_b2.reshape(1, 256),
                           w3k, enc_b3.reshape(1, 64))

    flatb = zTb.reshape(B * M, 64)
    cbt = codebook.T
    rn2 = rn.reshape(B * M, 1)

    idx_t = _distance_argmin(flatb, cbt, rn2)
    idx = idx_t.reshape(B * M)

    table = jnp.pad(codebook, ((0, 0), (0, 64)))
    rows = _sc_gather(table, idx)
    q = rows[:, :64].reshape(B, M, 64)

    w1b = dec_w1.astype(BF16)
    we1 = jnp.concatenate([w1b[:, :, 1], w1b[:, :, 3]], axis=0)
    wo1 = jnp.concatenate([w1b[:, :, 2], w1b[:, :, 0]], axis=0)
    w2b = dec_w2.astype(BF16)
    w2s = jnp.stack([
        jnp.concatenate([w2b[:, :, 1], w2b[:, :, 3]], axis=0),
        jnp.concatenate([w2b[:, :, 0], w2b[:, :, 2]], axis=0),
        jnp.concatenate([w2b[:, :, 1], w2b[:, :, 3]], axis=0),
        jnp.concatenate([w2b[:, :, 0], w2b[:, :, 2]], axis=0),
    ])
    w3v = dec_w3[:, 0, :].astype(BF16)
    zcol = jnp.zeros((128,), BF16)
    cols = []
    for s, pairs in enumerate([
        [(0, 1), (4, 3)], [(1, 0), (0, 2)], [(1, 1), (0, 3)],
        [(2, 0), (1, 2)], [(2, 1), (1, 3)], [(3, 0), (2, 2)],
        [(3, 1), (2, 3)], [(5, 0), (3, 2)],
    ]):
        blocks = [zcol] * 6
        for blk, k in pairs:
            blocks[blk] = w3v[:, k]
        cols.append(jnp.concatenate(blocks, axis=0))
    w3s = jnp.stack(cols, axis=1)

    qout, rec, loss = _decoder(q, zT, we1, wo1, dec_b1.reshape(1, 256),
                               w2s, dec_b2.reshape(1, 128),
                               w3s, dec_b3.reshape(1, 1))

    x_recon = rec.reshape(B, 1, 16384)
    vq_loss = loss.reshape(())
    quantized = qout.transpose(0, 2, 1)
    return x_recon, vq_loss, quantized
